# baseline (speedup 1.0000x reference)
import functools

import jax
import jax.numpy as jnp
from jax import lax
from jax.experimental import pallas as pl
from jax.experimental.pallas import tpu as pltpu
from jax.experimental.pallas import tpu_sc as plsc

S = 8192
B = 4
D = 1024
NC = 2
NSUB = 16
NW = NC * NSUB
S_PER_W = S // NW
NS_CHUNK = 8
ROWS = NS_CHUNK * B
N_CHUNKS = S_PER_W // NS_CHUNK
LANES = 16

_mesh = plsc.VectorSubcoreMesh(core_axis_name="c", subcore_axis_name="s")


@functools.partial(
    pl.kernel,
    mesh=_mesh,
    out_type=jax.ShapeDtypeStruct((S * B, D), jnp.float32),
    scratch_types=[
        pltpu.VMEM((S_PER_W * B,), jnp.int32),
        pltpu.VMEM((ROWS, D), jnp.float32),
        pltpu.VMEM((NS_CHUNK, D), jnp.float32),
        pltpu.SemaphoreType.DMA,
    ],
)
def _embed(x_hbm, tok_hbm, pos_hbm, out_hbm, idx_v, tok_v, pos_v, sem):
    wid = lax.axis_index("s") * NC + lax.axis_index("c")
    sbase = wid * S_PER_W
    rbase = sbase * B
    pltpu.sync_copy(x_hbm.at[pl.ds(rbase, S_PER_W * B)], idx_v)

    def chunk(g, carry):
        s0 = sbase + g * NS_CHUNK
        r0 = g * ROWS
        pltpu.async_copy(
            tok_hbm.at[idx_v.at[pl.ds(r0, ROWS)]], tok_v, sem
        ).wait()
        pltpu.sync_copy(pos_hbm.at[pl.ds(s0, NS_CHUNK)], pos_v)

        def srow(i, c2):
            def col(c, c3):
                sl = pl.ds(c * LANES, LANES)
                p = pos_v[i, sl]
                for b in range(B):
                    tok_v[i * B + b, sl] += p
                return c3

            return lax.fori_loop(0, D // LANES, col, c2)

        lax.fori_loop(0, NS_CHUNK, srow, 0)
        pltpu.sync_copy(tok_v, out_hbm.at[pl.ds(rbase + r0, ROWS)])
        return carry

    lax.fori_loop(0, N_CHUNKS, chunk, 0)


def kernel(x, token_table, pos_table):
    x_flat = x.reshape(-1)
    out = _embed(x_flat, token_table, pos_table)
    return out.reshape(S, B, D), x.shape[0]


# speedup vs baseline: 1.2978x; 1.2978x over previous
import functools

import jax
import jax.numpy as jnp
from jax import lax
from jax.experimental import pallas as pl
from jax.experimental.pallas import tpu as pltpu
from jax.experimental.pallas import tpu_sc as plsc

S = 8192
B = 4
D = 1024
NC = 2
NSUB = 16
NW = NC * NSUB
S_PER_W = S // NW
NS_CHUNK = 8
ROWS = NS_CHUNK * B
N_CHUNKS = S_PER_W // NS_CHUNK
LANES = 16
NBUF = 3

_mesh = plsc.VectorSubcoreMesh(core_axis_name="c", subcore_axis_name="s")


@functools.partial(
    pl.kernel,
    mesh=_mesh,
    out_type=jax.ShapeDtypeStruct((S * B, D), jnp.float32),
    scratch_types=[
        pltpu.VMEM((S_PER_W * B,), jnp.int32),
        pltpu.VMEM((NBUF, ROWS, D), jnp.float32),
        pltpu.VMEM((NBUF, NS_CHUNK, D), jnp.float32),
        pltpu.SemaphoreType.DMA((NBUF,)),
        pltpu.SemaphoreType.DMA((NBUF,)),
        pltpu.SemaphoreType.DMA((NBUF,)),
    ],
)
def _embed(x_hbm, tok_hbm, pos_hbm, out_hbm, idx_v, tok_v, pos_v,
           gsem, psem, osem):
    wid = lax.axis_index("s") * NC + lax.axis_index("c")
    sbase = wid * S_PER_W
    rbase = sbase * B
    pltpu.sync_copy(x_hbm.at[pl.ds(rbase, S_PER_W * B)], idx_v)

    def in_issue(g):
        b = g % NBUF
        pltpu.async_copy(
            tok_hbm.at[idx_v.at[pl.ds(g * ROWS, ROWS)]],
            tok_v.at[b], gsem.at[b])
        pltpu.async_copy(
            pos_hbm.at[pl.ds(sbase + g * NS_CHUNK, NS_CHUNK)],
            pos_v.at[b], psem.at[b])

    def in_wait(g):
        b = g % NBUF
        pltpu.make_async_copy(
            tok_hbm.at[idx_v.at[pl.ds(g * ROWS, ROWS)]],
            tok_v.at[b], gsem.at[b]).wait()
        pltpu.make_async_copy(
            pos_hbm.at[pl.ds(sbase + g * NS_CHUNK, NS_CHUNK)],
            pos_v.at[b], psem.at[b]).wait()

    def out_issue(g):
        b = g % NBUF
        pltpu.async_copy(
            tok_v.at[b], out_hbm.at[pl.ds(rbase + g * ROWS, ROWS)],
            osem.at[b])

    def out_wait(g):
        b = g % NBUF
        pltpu.make_async_copy(
            tok_v.at[b], out_hbm.at[pl.ds(rbase + g * ROWS, ROWS)],
            osem.at[b]).wait()

    def add_chunk(g):
        b = g % NBUF
        tok_s = tok_v.at[b]
        pos_s = pos_v.at[b]

        def srow(i, c2):
            def col(c, c3):
                sl = pl.ds(c * LANES, LANES)
                p = pos_s[i, sl]
                for bb in range(B):
                    tok_s[i * B + bb, sl] += p
                return c3

            return lax.fori_loop(0, D // LANES, col, c2)

        lax.fori_loop(0, NS_CHUNK, srow, 0)

    for g in range(NBUF - 1):
        in_issue(g)
    for g in range(N_CHUNKS):
        if g + NBUF - 1 < N_CHUNKS:
            if g - 1 >= 0:
                out_wait(g - 1)
            in_issue(g + NBUF - 1)
        in_wait(g)
        add_chunk(g)
        out_issue(g)
    for g in range(N_CHUNKS - NBUF, N_CHUNKS):
        out_wait(g)


def kernel(x, token_table, pos_table):
    x_flat = x.reshape(-1)
    out = _embed(x_flat, token_table, pos_table)
    return out.reshape(S, B, D), x.shape[0]


# speedup vs baseline: 1.4760x; 1.1373x over previous
import functools

import jax
import jax.numpy as jnp
from jax import lax
from jax.experimental import pallas as pl
from jax.experimental.pallas import tpu as pltpu
from jax.experimental.pallas import tpu_sc as plsc

S = 8192
B = 4
D = 1024
NC = 2
NSUB = 16
NW = NC * NSUB
S_PER_W = S // NW
NS_CHUNK = 8
ROWS = NS_CHUNK * B
N_CHUNKS = S_PER_W // NS_CHUNK
LANES = 16
NBUF = 3

_mesh = plsc.VectorSubcoreMesh(core_axis_name="c", subcore_axis_name="s")


@functools.partial(
    pl.kernel,
    mesh=_mesh,
    out_type=jax.ShapeDtypeStruct((S * B, D), jnp.float32),
    scratch_types=[
        pltpu.VMEM((S_PER_W * B,), jnp.int32),
        pltpu.VMEM((NBUF, ROWS, D), jnp.float32),
        pltpu.VMEM((NBUF, NS_CHUNK, D), jnp.float32),
        pltpu.SemaphoreType.DMA((NBUF,)),
        pltpu.SemaphoreType.DMA((NBUF,)),
        pltpu.SemaphoreType.DMA((NBUF,)),
    ],
)
def _embed(x_hbm, tok_hbm, pos_hbm, out_hbm, idx_v, tok_v, pos_v,
           gsem, psem, osem):
    wid = lax.axis_index("s") * NC + lax.axis_index("c")
    sbase = wid * S_PER_W
    rbase = sbase * B
    pltpu.sync_copy(x_hbm.at[pl.ds(rbase, S_PER_W * B)], idx_v)

    def in_issue(g):
        b = g % NBUF
        pltpu.async_copy(
            tok_hbm.at[idx_v.at[pl.ds(g * ROWS, ROWS)]],
            tok_v.at[b], gsem.at[b])
        pltpu.async_copy(
            pos_hbm.at[pl.ds(sbase + g * NS_CHUNK, NS_CHUNK)],
            pos_v.at[b], psem.at[b])

    def in_wait(g):
        b = g % NBUF
        pltpu.make_async_copy(
            tok_hbm.at[idx_v.at[pl.ds(g * ROWS, ROWS)]],
            tok_v.at[b], gsem.at[b]).wait()
        pltpu.make_async_copy(
            pos_hbm.at[pl.ds(sbase + g * NS_CHUNK, NS_CHUNK)],
            pos_v.at[b], psem.at[b]).wait()

    def out_issue(g):
        b = g % NBUF
        pltpu.async_copy(
            tok_v.at[b], out_hbm.at[pl.ds(rbase + g * ROWS, ROWS)],
            osem.at[b])

    def out_wait(g):
        b = g % NBUF
        pltpu.make_async_copy(
            tok_v.at[b], out_hbm.at[pl.ds(rbase + g * ROWS, ROWS)],
            osem.at[b]).wait()

    def add_chunk(g):
        b = g % NBUF
        tok_s = tok_v.at[b]
        pos_s = pos_v.at[b]

        def col(c, c3):
            sl = pl.ds(c * LANES, LANES)
            for i in range(NS_CHUNK):
                p = pos_s[i, sl]
                for bb in range(B):
                    tok_s[i * B + bb, sl] += p
            return c3

        lax.fori_loop(0, D // LANES, col, 0)

    for g in range(NBUF - 1):
        in_issue(g)
    for g in range(N_CHUNKS):
        if g + NBUF - 1 < N_CHUNKS:
            if g - 1 >= 0:
                out_wait(g - 1)
            in_issue(g + NBUF - 1)
        in_wait(g)
        add_chunk(g)
        out_issue(g)
    for g in range(N_CHUNKS - NBUF, N_CHUNKS):
        out_wait(g)


def kernel(x, token_table, pos_table):
    x_flat = x.reshape(-1)
    out = _embed(x_flat, token_table, pos_table)
    return out.reshape(S, B, D), x.shape[0]
